# baseline (speedup 1.0000x reference)
import functools

import numpy as np
import jax
import jax.numpy as jnp
from jax import lax
from jax.experimental import pallas as pl
from jax.experimental.pallas import tpu as pltpu
from jax.experimental.pallas import tpu_sc as plsc

B = 100000
D = 11
G = 10
LANES = 16

NUM_CORES = 2
NUM_SUBCORES = 16
NW = NUM_CORES * NUM_SUBCORES

CHUNK = 512
NCHUNK = -(-B // CHUNK)
LAST_BASE = B - CHUNK
VPC = CHUNK // LANES
MAX_CHUNKS = -(-NCHUNK // NW)


def _sq_thresholds():
    out = []
    for t in (0.1, 0.2, 0.3, 0.4, 0.5, 0.6, 0.7, 0.8, 0.9):
        t32 = np.float32(t)
        s = np.float32(t32 * t32)
        while np.float32(np.sqrt(np.nextafter(s, np.float32(0)))) >= t32:
            s = np.nextafter(s, np.float32(0))
        while np.float32(np.sqrt(s)) < t32:
            s = np.nextafter(s, np.float32(np.inf))
        out.append(float(s))
    return tuple(out)


_SQ_T = _sq_thresholds()


def _build():
    mesh = plsc.VectorSubcoreMesh(core_axis_name="c", subcore_axis_name="s")

    @functools.partial(
        pl.kernel,
        out_type=jax.ShapeDtypeStruct((B,), jnp.float32),
        mesh=mesh,
        compiler_params=pltpu.CompilerParams(needs_layout_passes=False),
        scratch_types=[
            pltpu.VMEM((CHUNK * D,), jnp.float32),
            pltpu.VMEM((CHUNK * D,), jnp.float32),
            pltpu.VMEM((CHUNK * G,), jnp.float32),
            pltpu.VMEM((CHUNK,), jnp.float32),
        ],
    )
    def k(z1_hbm, z2_hbm, dg_hbm, out_hbm, z1_v, z2_v, dg_v, out_v):
        wid = lax.axis_index("s") * NUM_CORES + lax.axis_index("c")
        lane = lax.iota(jnp.int32, LANES)

        def chunk_body(i, carry):
            c = wid + i * NW

            @pl.when(c < NCHUNK)
            def _():
                base = jnp.minimum(c * CHUNK, LAST_BASE)
                pltpu.sync_copy(z1_hbm.at[pl.ds(base * D, CHUNK * D)], z1_v)
                pltpu.sync_copy(z2_hbm.at[pl.ds(base * D, CHUNK * D)], z2_v)
                pltpu.sync_copy(dg_hbm.at[pl.ds(base * G, CHUNK * G)], dg_v)

                def vec_body(v, carry2):
                    rows = lane + v * LANES
                    zi = rows * D
                    x1 = plsc.load_gather(z1_v, [zi])
                    y1 = plsc.load_gather(z1_v, [zi + 2])
                    x2 = plsc.load_gather(z2_v, [zi])
                    y2 = plsc.load_gather(z2_v, [zi + 2])
                    dx = x2 - x1
                    dy = y2 - y1
                    s = dx * dx + dy * dy
                    did = jnp.zeros((LANES,), jnp.int32)
                    for thr in _SQ_T:
                        did = did + (s >= jnp.float32(thr)).astype(jnp.int32)
                    g = plsc.load_gather(dg_v, [rows * G + did])
                    out_v[pl.ds(v * LANES, LANES)] = g
                    return carry2

                lax.fori_loop(0, VPC, vec_body, None)
                pltpu.sync_copy(out_v, out_hbm.at[pl.ds(base, CHUNK)])

            return carry

        lax.fori_loop(0, MAX_CHUNKS, chunk_body, None)

    return k


_sc_kernel = _build()


def kernel(z_1, z_2, dist_grade):
    return _sc_kernel(
        z_1.reshape(-1), z_2.reshape(-1), dist_grade.reshape(-1)
    )


# speedup vs baseline: 1.0595x; 1.0595x over previous
import functools

import numpy as np
import jax
import jax.numpy as jnp
from jax import lax
from jax.experimental import pallas as pl
from jax.experimental.pallas import tpu as pltpu
from jax.experimental.pallas import tpu_sc as plsc

B = 100000
D = 11
G = 10
LANES = 16

NUM_CORES = 2
NUM_SUBCORES = 16
NW = NUM_CORES * NUM_SUBCORES

CHUNK = 512
NCHUNK = -(-B // CHUNK)
LAST_BASE = B - CHUNK
VPC = CHUNK // LANES
MAX_CHUNKS = -(-NCHUNK // NW)
UNROLL = 4


def _sq_thresholds():
    out = []
    for t in (0.1, 0.2, 0.3, 0.4, 0.5, 0.6, 0.7, 0.8, 0.9):
        t32 = np.float32(t)
        s = np.float32(t32 * t32)
        while np.float32(np.sqrt(np.nextafter(s, np.float32(0)))) >= t32:
            s = np.nextafter(s, np.float32(0))
        while np.float32(np.sqrt(s)) < t32:
            s = np.nextafter(s, np.float32(np.inf))
        out.append(float(s))
    return tuple(out)


_SQ_T = _sq_thresholds()


def _build():
    mesh = plsc.VectorSubcoreMesh(core_axis_name="c", subcore_axis_name="s")

    @functools.partial(
        pl.kernel,
        out_type=jax.ShapeDtypeStruct((B,), jnp.float32),
        mesh=mesh,
        compiler_params=pltpu.CompilerParams(needs_layout_passes=False),
        scratch_types=[
            pltpu.VMEM((CHUNK * D,), jnp.float32),
            pltpu.VMEM((CHUNK * D,), jnp.float32),
            pltpu.VMEM((CHUNK * D,), jnp.float32),
            pltpu.VMEM((CHUNK * D,), jnp.float32),
            pltpu.VMEM((CHUNK * G,), jnp.float32),
            pltpu.VMEM((CHUNK * G,), jnp.float32),
            pltpu.VMEM((CHUNK,), jnp.float32),
            pltpu.VMEM((CHUNK,), jnp.float32),
            pltpu.SemaphoreType.DMA,
            pltpu.SemaphoreType.DMA,
            pltpu.SemaphoreType.DMA,
            pltpu.SemaphoreType.DMA,
        ],
    )
    def k(z1_hbm, z2_hbm, dg_hbm, out_hbm,
          z1a, z1b, z2a, z2b, dga, dgb, outa, outb,
          isem_a, isem_b, osem_a, osem_b):
        z1buf = (z1a, z1b)
        z2buf = (z2a, z2b)
        dgbuf = (dga, dgb)
        outbuf = (outa, outb)
        isem = (isem_a, isem_b)
        osem = (osem_a, osem_b)

        wid = lax.axis_index("s") * NUM_CORES + lax.axis_index("c")
        lane = lax.iota(jnp.int32, LANES)

        def chunk_base(i):
            return jnp.minimum((wid + i * NW) * CHUNK, LAST_BASE)

        def in_flight(i):
            return wid + i * NW < NCHUNK

        def start_in(i):
            @pl.when(in_flight(i))
            def _():
                b = i % 2
                base = chunk_base(i)
                pltpu.async_copy(
                    z1_hbm.at[pl.ds(base * D, CHUNK * D)], z1buf[b], isem[b])
                pltpu.async_copy(
                    z2_hbm.at[pl.ds(base * D, CHUNK * D)], z2buf[b], isem[b])
                pltpu.async_copy(
                    dg_hbm.at[pl.ds(base * G, CHUNK * G)], dgbuf[b], isem[b])

        def drain_in(i):
            b = i % 2
            base = chunk_base(i)
            pltpu.make_async_copy(
                z1_hbm.at[pl.ds(base * D, CHUNK * D)], z1buf[b], isem[b]).wait()
            pltpu.make_async_copy(
                z2_hbm.at[pl.ds(base * D, CHUNK * D)], z2buf[b], isem[b]).wait()
            pltpu.make_async_copy(
                dg_hbm.at[pl.ds(base * G, CHUNK * G)], dgbuf[b], isem[b]).wait()

        def drain_out(i):
            @pl.when(in_flight(i))
            def _():
                b = i % 2
                base = chunk_base(i)
                pltpu.make_async_copy(
                    outbuf[b], out_hbm.at[pl.ds(base, CHUNK)], osem[b]).wait()

        def compute_vec(b, v):
            rows = lane + v * LANES
            zi = rows * D
            x1 = plsc.load_gather(z1buf[b], [zi])
            y1 = plsc.load_gather(z1buf[b], [zi + 2])
            x2 = plsc.load_gather(z2buf[b], [zi])
            y2 = plsc.load_gather(z2buf[b], [zi + 2])
            dx = x2 - x1
            dy = y2 - y1
            s = dx * dx + dy * dy
            did = jnp.zeros((LANES,), jnp.int32)
            for thr in _SQ_T:
                did = did + (s >= jnp.float32(thr)).astype(jnp.int32)
            g = plsc.load_gather(dgbuf[b], [rows * G + did])
            outbuf[b][pl.ds(v * LANES, LANES)] = g

        start_in(0)
        for i in range(MAX_CHUNKS):
            start_in(i + 1) if i + 1 < MAX_CHUNKS else None
            if i >= 2:
                drain_out(i - 2)

            @pl.when(in_flight(i))
            def _(i=i):
                b = i % 2
                drain_in(i)

                def vec_body(u, carry):
                    for j in range(UNROLL):
                        compute_vec(b, u * UNROLL + j)
                    return carry

                lax.fori_loop(0, VPC // UNROLL, vec_body, None)
                pltpu.async_copy(
                    outbuf[b], out_hbm.at[pl.ds(chunk_base(i), CHUNK)], osem[b])

        for i in range(max(0, MAX_CHUNKS - 2), MAX_CHUNKS):
            drain_out(i)

    return k


_sc_kernel = _build()


def kernel(z_1, z_2, dist_grade):
    return _sc_kernel(
        z_1.reshape(-1), z_2.reshape(-1), dist_grade.reshape(-1)
    )


# speedup vs baseline: 7.8286x; 7.3892x over previous
import functools

import numpy as np
import jax
import jax.numpy as jnp
from jax import lax
from jax.experimental import pallas as pl
from jax.experimental.pallas import tpu as pltpu
from jax.experimental.pallas import tpu_sc as plsc

B = 100000
D = 11
G = 10
LANES = 16

NUM_CORES = 2
NUM_SUBCORES = 16
NW = NUM_CORES * NUM_SUBCORES

CHUNK = 512
NFULL = 195
EXTRA_BASE = NFULL * CHUNK
EXTRA = 128
TAIL_BASE = EXTRA_BASE + EXTRA
TAIL = B - TAIL_BASE
MAX_CHUNKS = NFULL // NW + 1
UNROLL = 4

EXTRA_WID = 3
TAIL_WID = 4


def _sq_thresholds():
    out = []
    for t in (0.1, 0.2, 0.3, 0.4, 0.5, 0.6, 0.7, 0.8, 0.9):
        t32 = np.float32(t)
        s = np.float32(t32 * t32)
        while np.float32(np.sqrt(np.nextafter(s, np.float32(0)))) >= t32:
            s = np.nextafter(s, np.float32(0))
        while np.float32(np.sqrt(s)) < t32:
            s = np.nextafter(s, np.float32(np.inf))
        out.append(float(s))
    return tuple(out)


_SQ_T = _sq_thresholds()


def _build():
    mesh = plsc.VectorSubcoreMesh(core_axis_name="c", subcore_axis_name="s")

    @functools.partial(
        pl.kernel,
        out_type=jax.ShapeDtypeStruct((B,), jnp.float32),
        mesh=mesh,
        compiler_params=pltpu.CompilerParams(
            needs_layout_passes=False, skip_device_barrier=True),
        scratch_types=[
            pltpu.VMEM((8, CHUNK), jnp.float32),
            pltpu.VMEM((8, CHUNK), jnp.float32),
            pltpu.VMEM((8, CHUNK), jnp.float32),
            pltpu.VMEM((8, CHUNK), jnp.float32),
            pltpu.VMEM((G, CHUNK), jnp.float32),
            pltpu.VMEM((G, CHUNK), jnp.float32),
            pltpu.VMEM((CHUNK,), jnp.float32),
            pltpu.VMEM((CHUNK,), jnp.float32),
            pltpu.VMEM((8, TAIL), jnp.float32),
            pltpu.VMEM((8, TAIL), jnp.float32),
            pltpu.VMEM((G, TAIL), jnp.float32),
            pltpu.VMEM((TAIL,), jnp.float32),
            pltpu.SemaphoreType.DMA,
            pltpu.SemaphoreType.DMA,
            pltpu.SemaphoreType.DMA,
            pltpu.SemaphoreType.DMA,
        ],
    )
    def k(z1_hbm, z2_hbm, dg_hbm, out_hbm,
          z1a, z1b, z2a, z2b, dga, dgb, outa, outb,
          z1t, z2t, dgt, outt,
          isem_a, isem_b, osem_a, osem_b):
        z1buf = (z1a, z1b)
        z2buf = (z2a, z2b)
        dgbuf = (dga, dgb)
        outbuf = (outa, outb)
        isem = (isem_a, isem_b)
        osem = (osem_a, osem_b)

        wid = lax.axis_index("s") * NUM_CORES + lax.axis_index("c")
        lane = lax.iota(jnp.int32, LANES)

        def full_base(i):
            return pl.multiple_of((wid + i * NW) * CHUNK, 128)

        def in_copies(i):
            b = i % 2
            base = full_base(i)
            return (
                pltpu.make_async_copy(
                    z1_hbm.at[pl.ds(0, 8), pl.ds(base, CHUNK)],
                    z1buf[b], isem[b]),
                pltpu.make_async_copy(
                    z2_hbm.at[pl.ds(0, 8), pl.ds(base, CHUNK)],
                    z2buf[b], isem[b]),
                pltpu.make_async_copy(
                    dg_hbm.at[:, pl.ds(base, CHUNK)],
                    dgbuf[b], isem[b]),
            )

        def extra_in_copies():
            b = (MAX_CHUNKS - 1) % 2
            return (
                pltpu.make_async_copy(
                    z1_hbm.at[pl.ds(0, 8), pl.ds(EXTRA_BASE, EXTRA)],
                    z1buf[b].at[:, pl.ds(0, EXTRA)], isem[b]),
                pltpu.make_async_copy(
                    z2_hbm.at[pl.ds(0, 8), pl.ds(EXTRA_BASE, EXTRA)],
                    z2buf[b].at[:, pl.ds(0, EXTRA)], isem[b]),
                pltpu.make_async_copy(
                    dg_hbm.at[:, pl.ds(EXTRA_BASE, EXTRA)],
                    dgbuf[b].at[:, pl.ds(0, EXTRA)], isem[b]),
            )

        def tail_in_copies():
            b = (MAX_CHUNKS - 1) % 2
            return (
                pltpu.make_async_copy(
                    z1_hbm.at[pl.ds(0, 8), pl.ds(TAIL_BASE, TAIL)],
                    z1t, isem[b]),
                pltpu.make_async_copy(
                    z2_hbm.at[pl.ds(0, 8), pl.ds(TAIL_BASE, TAIL)],
                    z2t, isem[b]),
                pltpu.make_async_copy(
                    dg_hbm.at[:, pl.ds(TAIL_BASE, TAIL)],
                    dgt, isem[b]),
            )

        def out_copy(i):
            b = i % 2
            return pltpu.make_async_copy(
                outbuf[b], out_hbm.at[pl.ds(full_base(i), CHUNK)], osem[b])

        def extra_out_copy():
            b = (MAX_CHUNKS - 1) % 2
            return pltpu.make_async_copy(
                outbuf[b].at[pl.ds(0, EXTRA)],
                out_hbm.at[pl.ds(EXTRA_BASE, EXTRA)], osem[b])

        def tail_out_copy():
            b = (MAX_CHUNKS - 1) % 2
            return pltpu.make_async_copy(
                outt, out_hbm.at[pl.ds(TAIL_BASE, TAIL)], osem[b])

        def start_in(i):
            if i < MAX_CHUNKS - 1:
                for c in in_copies(i):
                    c.start()
            else:
                @pl.when(wid < NFULL - (MAX_CHUNKS - 1) * NW)
                def _():
                    for c in in_copies(i):
                        c.start()

                @pl.when(wid == EXTRA_WID)
                def _():
                    for c in extra_in_copies():
                        c.start()

                @pl.when(wid == TAIL_WID)
                def _():
                    for c in tail_in_copies():
                        c.start()

        def compute_vec(z1r, z2r, dgr, outr, off):
            x1 = z1r[0, pl.ds(off, LANES)]
            y1 = z1r[2, pl.ds(off, LANES)]
            x2 = z2r[0, pl.ds(off, LANES)]
            y2 = z2r[2, pl.ds(off, LANES)]
            dx = x2 - x1
            dy = y2 - y1
            s = dx * dx + dy * dy
            did = jnp.zeros((LANES,), jnp.int32)
            for thr in _SQ_T:
                did = did + (s >= jnp.float32(thr)).astype(jnp.int32)
            g = plsc.load_gather(dgr, [did, lane + off])
            outr[pl.ds(off, LANES)] = g

        def compute(i):
            b = i % 2
            for c in in_copies(i):
                c.wait()

            def vec_body(u, carry):
                for j in range(UNROLL):
                    compute_vec(z1buf[b], z2buf[b], dgbuf[b], outbuf[b],
                                (u * UNROLL + j) * LANES)
                return carry

            lax.fori_loop(0, CHUNK // LANES // UNROLL, vec_body, None)
            out_copy(i).start()

        def compute_extra():
            b = (MAX_CHUNKS - 1) % 2
            for c in extra_in_copies():
                c.wait()
            for v in range(EXTRA // LANES):
                compute_vec(z1buf[b], z2buf[b], dgbuf[b], outbuf[b],
                            v * LANES)
            extra_out_copy().start()

        def compute_tail():
            for c in tail_in_copies():
                c.wait()
            for v in range(TAIL // LANES):
                compute_vec(z1t, z2t, dgt, outt, v * LANES)
            tail_out_copy().start()

        start_in(0)
        for i in range(MAX_CHUNKS):
            if i + 1 < MAX_CHUNKS:
                start_in(i + 1)
            if i >= 2:
                out_copy(i - 2).wait()
            if i < MAX_CHUNKS - 1:
                compute(i)
            else:
                pl.when(wid < NFULL - i * NW)(lambda: compute(i))
                pl.when(wid == EXTRA_WID)(compute_extra)
                pl.when(wid == TAIL_WID)(compute_tail)

        out_copy(MAX_CHUNKS - 2).wait()
        last = MAX_CHUNKS - 1
        pl.when(wid < NFULL - last * NW)(lambda: out_copy(last).wait())
        pl.when(wid == EXTRA_WID)(lambda: extra_out_copy().wait())
        pl.when(wid == TAIL_WID)(lambda: tail_out_copy().wait())

    return k


_sc_kernel = _build()


def kernel(z_1, z_2, dist_grade):
    return _sc_kernel(z_1.T, z_2.T, dist_grade.T)
